# Initial kernel scaffold; baseline (speedup 1.0000x reference)
import jax
import jax.numpy as jnp
from jax.experimental import pallas as pl


def kernel(enc_g2c_src, enc_g2c_dst, enc_c2g_src, enc_c2g_dst, dec_src, dec_dst, gene_emb, cell_emb, W_g2c, b_g2c, W_c2g, b_c2g, Wp, bp):
    raise NotImplementedError("write your pallas kernel here")


# speedup vs baseline: 2.2696x; 2.2696x over previous
import functools

import jax
import jax.numpy as jnp
from jax import lax
from jax.experimental import pallas as pl
from jax.experimental.pallas import tpu as pltpu
from jax.experimental.pallas import tpu_sc as plsc

NG = 4762
NC = 847
D = 256
E_ENC = 150000
E_DEC = 200000

GP = 4768
CP = 864

NCORES = 2
NSUB = 16
NW = NCORES * NSUB

WIN = 128
CH_ENC = 4736
NWIN = CH_ENC // WIN
E_ENC_PAD = NW * CH_ENC

CH_DEC = 6272
E_DEC_PAD = NW * CH_DEC

f32 = jnp.float32
i32 = jnp.int32


SEG = 2 * GP + 2 * CP


def _deg_body(g2cs, g2cd, c2gs, c2gd, deg_out,
              ci_v, cgo, cci, cco, cgi):
    c = lax.axis_index("c")
    s = lax.axis_index("s")
    k = c * NSUB + s
    ones16 = jnp.ones((16,), f32)

    for buf, n in ((cgo, GP), (cci, CP), (cco, CP), (cgi, GP)):
        def zb(i, _, buf=buf):
            buf[pl.ds(i * 16, 16)] = jnp.zeros((16,), f32)
            return 0
        lax.fori_loop(0, n // 16, zb, 0)

    for idx_hbm, cnt in ((g2cs, cgo), (g2cd, cci), (c2gs, cco), (c2gd, cgi)):
        pltpu.sync_copy(idx_hbm.at[pl.ds(k * CH_ENC, CH_ENC)], ci_v)

        def b(i, _, cnt=cnt):
            ii = ci_v[pl.ds(i * 16, 16)]
            plsc.addupdate_scatter(cnt, [ii], ones16)
            return 0
        lax.fori_loop(0, CH_ENC // 16, b, 0)

    base = k * SEG
    pltpu.sync_copy(cgo, deg_out.at[pl.ds(base, GP)])
    pltpu.sync_copy(cci, deg_out.at[pl.ds(base + GP, CP)])
    pltpu.sync_copy(cco, deg_out.at[pl.ds(base + GP + CP, CP)])
    pltpu.sync_copy(cgi, deg_out.at[pl.ds(base + GP + 2 * CP, GP)])


def _scale_body(gp_ref, cp_ref, deg_ref, xg_ref, xc_ref, ds_ref):
    dsum = jnp.sum(deg_ref[...], axis=0)
    dgo = lax.slice(dsum, (0,), (GP,))
    dco = lax.slice(dsum, (GP + CP,), (GP + 2 * CP,))
    rg = lax.rsqrt(jnp.clip(dgo, 1.0, None))
    rc = lax.rsqrt(jnp.clip(dco, 1.0, None))
    xg_ref[...] = gp_ref[...] * rg[:, None]
    xc_ref[...] = cp_ref[...] * rc[:, None]
    ds_ref[...] = dsum


_scale_call = pl.pallas_call(
    _scale_body,
    out_shape=[
        jax.ShapeDtypeStruct((GP, D), f32),
        jax.ShapeDtypeStruct((CP, D), f32),
        jax.ShapeDtypeStruct((SEG,), f32),
    ],
)


def _gat_body(srcidx, tab, out, sidx_v, msg_v, sem):
    c = lax.axis_index("c")
    s = lax.axis_index("s")
    k = c * NSUB + s

    def win_body(w, _):
        base = k * CH_ENC + w * WIN
        pltpu.sync_copy(srcidx.at[pl.ds(base, WIN)], sidx_v.at[0])
        pltpu.async_copy(tab.at[sidx_v.at[0]], msg_v, sem).wait()
        pltpu.sync_copy(msg_v, out.at[pl.ds(base, WIN)])
        return 0
    lax.fori_loop(0, NWIN, win_body, 0)


def _head_body(aggc_ref, aggg_ref, ds_ref,
               wgc_ref, bgc_ref, wcg_ref, bcg_ref,
               wpg_ref, wpc_ref, bp_ref,
               hg_ref, hc_ref, sg_ref, sc_ref):
    dsum = ds_ref[...]
    dci = lax.slice(dsum, (GP,), (GP + CP,))
    dgi = lax.slice(dsum, (GP + 2 * CP,), (SEG,))
    ric = lax.rsqrt(jnp.clip(dci, 1.0, None))
    rig = lax.rsqrt(jnp.clip(dgi, 1.0, None))
    aggc = aggc_ref[...] * ric[:, None]
    aggg = aggg_ref[...] * rig[:, None]
    h_cell = jnp.maximum(
        jnp.dot(aggc, wgc_ref[...], preferred_element_type=f32)
        + bgc_ref[...], 0.0)
    h_gene = jnp.maximum(
        jnp.dot(aggg, wcg_ref[...], preferred_element_type=f32)
        + bcg_ref[...], 0.0)
    hg_ref[...] = h_gene
    hc_ref[...] = h_cell
    sg_ref[...] = jnp.dot(h_gene, wpg_ref[...], preferred_element_type=f32)
    sc_ref[...] = (jnp.dot(h_cell, wpc_ref[...], preferred_element_type=f32)
                   + bp_ref[0, 0])


_head_call = pl.pallas_call(
    _head_body,
    out_shape=[
        jax.ShapeDtypeStruct((GP, D), f32),
        jax.ShapeDtypeStruct((CP, D), f32),
        jax.ShapeDtypeStruct((GP, 1), f32),
        jax.ShapeDtypeStruct((CP, 1), f32),
    ],
)


def _dec_body(sg_hbm, sc_hbm, dsrc, ddst, score_out,
              sg_v, sc_v, si_v, di_v, out_v):
    c = lax.axis_index("c")
    s = lax.axis_index("s")
    k = c * NSUB + s
    base = k * CH_DEC

    pltpu.sync_copy(sg_hbm, sg_v)
    pltpu.sync_copy(sc_hbm, sc_v)
    pltpu.sync_copy(dsrc.at[pl.ds(base, CH_DEC)], si_v)
    pltpu.sync_copy(ddst.at[pl.ds(base, CH_DEC)], di_v)

    def body(i, _):
        a = plsc.load_gather(sg_v, [si_v[pl.ds(i * 16, 16)]])
        b = plsc.load_gather(sc_v, [di_v[pl.ds(i * 16, 16)]])
        out_v[pl.ds(i * 16, 16)] = a + b
        return 0

    lax.fori_loop(0, CH_DEC // 16, body, 0)

    pltpu.sync_copy(out_v, score_out.at[pl.ds(base, CH_DEC)])


@functools.cache
def _sc_kernels():
    mesh = plsc.VectorSubcoreMesh(
        core_axis_name="c", subcore_axis_name="s", num_cores=NCORES)
    deg = pl.kernel(
        _deg_body,
        out_type=jax.ShapeDtypeStruct((NW * SEG,), f32),
        mesh=mesh,
        compiler_params=pltpu.CompilerParams(needs_layout_passes=False),
        scratch_types=[
            pltpu.VMEM((CH_ENC,), i32),
            pltpu.VMEM((GP,), f32),
            pltpu.VMEM((CP,), f32),
            pltpu.VMEM((CP,), f32),
            pltpu.VMEM((GP,), f32),
        ],
    )
    gat = pl.kernel(
        _gat_body,
        out_type=jax.ShapeDtypeStruct((E_ENC_PAD, D), f32),
        mesh=mesh,
        scratch_types=[
            pltpu.VMEM((1, WIN), i32),
            pltpu.VMEM((WIN, D), f32),
            pltpu.SemaphoreType.DMA,
        ],
    )
    dec = pl.kernel(
        _dec_body,
        out_type=jax.ShapeDtypeStruct((E_DEC_PAD,), f32),
        mesh=mesh,
        compiler_params=pltpu.CompilerParams(needs_layout_passes=False),
        scratch_types=[
            pltpu.VMEM((GP,), f32),
            pltpu.VMEM((CP,), f32),
            pltpu.VMEM((CH_DEC,), i32),
            pltpu.VMEM((CH_DEC,), i32),
            pltpu.VMEM((CH_DEC,), f32),
        ],
    )
    return deg, gat, dec


def kernel(enc_g2c_src, enc_g2c_dst, enc_c2g_src, enc_c2g_dst,
           dec_src, dec_dst,
           gene_emb, cell_emb, W_g2c, b_g2c, W_c2g, b_c2g, Wp, bp):
    pad_n = E_ENC_PAD - E_ENC
    padg = (jnp.arange(pad_n, dtype=i32) % (GP - NG)) + NG
    padc = (jnp.arange(pad_n, dtype=i32) % (CP - NC)) + NC
    g2cs = jnp.concatenate([enc_g2c_src.astype(i32), padg])
    g2cd = jnp.concatenate([enc_g2c_dst.astype(i32), padc])
    c2gs = jnp.concatenate([enc_c2g_src.astype(i32), padc])
    c2gd = jnp.concatenate([enc_c2g_dst.astype(i32), padg])

    padd = jnp.zeros((E_DEC_PAD - E_DEC,), i32)
    dsrc = jnp.concatenate([dec_src.astype(i32), padd])
    ddst = jnp.concatenate([dec_dst.astype(i32), padd])

    gp = jnp.zeros((GP, D), f32).at[:NG].set(gene_emb)
    cp = jnp.zeros((CP, D), f32).at[:NC].set(cell_emb)

    deg_k, gat_k, dec_k = _sc_kernels()
    degall = deg_k(g2cs, g2cd, c2gs, c2gd).reshape(NW, SEG)
    xg, xc, dsum = _scale_call(gp, cp, degall)

    msgs_g = gat_k(g2cs, xg)
    msgs_c = gat_k(c2gs, xc)
    aggc = jnp.zeros((CP, D), f32).at[g2cd].add(msgs_g)
    aggg = jnp.zeros((GP, D), f32).at[c2gd].add(msgs_c)

    hg, hc, sg, sc = _head_call(
        aggc, aggg, dsum,
        W_g2c, b_g2c.reshape(1, D), W_c2g, b_c2g.reshape(1, D),
        Wp[:D], Wp[D:], bp.reshape(1, 1))
    score = dec_k(sg.reshape(GP), sc.reshape(CP), dsrc, ddst)

    return (score[:E_DEC].reshape(E_DEC, 1), hg[:NG], hc[:NC])
